# baseline (speedup 1.0000x reference)
import functools

import jax
import jax.numpy as jnp
from jax import lax
from jax.experimental import pallas as pl
from jax.experimental.pallas import tpu as pltpu
from jax.experimental.pallas import tpu_sc as plsc

_NC = 2
_NS = 16
_NW = _NC * _NS
_CH = 128
_PT = 102
_NBUF = 6
_TOT = _NW * _PT * _CH

_R = 32


def _node_body(x_ref, wn_ref, bn_ref, wt_ref, bt_ref, ux_ref, vx_ref):
    xb = x_ref[...]
    ux_ref[...] = (
        jnp.dot(xb, wn_ref[...], preferred_element_type=jnp.float32) + bn_ref[...]
    )
    vx_ref[...] = (
        jnp.dot(xb, wt_ref[...], preferred_element_type=jnp.float32) + bt_ref[...]
    )


def _node_embeddings(x2, wn_t, bn, wt_t, bt):
    bn_rows, h = x2.shape
    blk = 2000 if bn_rows % 2000 == 0 else bn_rows
    grid = bn_rows // blk
    return pl.pallas_call(
        _node_body,
        grid=(grid,),
        in_specs=[
            pl.BlockSpec((blk, h), lambda i: (i, 0)),
            pl.BlockSpec((h, h), lambda i: (0, 0)),
            pl.BlockSpec((1, h), lambda i: (0, 0)),
            pl.BlockSpec((h, h), lambda i: (0, 0)),
            pl.BlockSpec((1, h), lambda i: (0, 0)),
        ],
        out_specs=[
            pl.BlockSpec((blk, h), lambda i: (i, 0)),
            pl.BlockSpec((blk, h), lambda i: (i, 0)),
        ],
        out_shape=[
            jax.ShapeDtypeStruct((bn_rows, h), jnp.float32),
            jax.ShapeDtypeStruct((bn_rows, h), jnp.float32),
        ],
    )(x2, wn_t, bn, wt_t, bt)


def _sc_gather(table, idx3):
    hw = table.shape[1]
    dt = table.dtype
    mesh = plsc.VectorSubcoreMesh(
        core_axis_name="c", subcore_axis_name="s", num_cores=_NC, num_subcores=_NS
    )

    @functools.partial(
        pl.kernel,
        out_type=jax.ShapeDtypeStruct((_NW * _PT * _CH, hw), dt),
        mesh=mesh,
        scratch_types=[
            pltpu.VMEM((_PT, _CH), jnp.int32),
            pltpu.VMEM((_NBUF, _CH, hw), dt),
            pltpu.SemaphoreType.DMA((_NBUF,)),
            pltpu.SemaphoreType.DMA((_NBUF,)),
        ],
    )
    def gather_k(table_hbm, idx_hbm, out_hbm, idx_v, rows_v, gsem, ssem):
        wid = lax.axis_index("s") * _NC + lax.axis_index("c")
        base = wid * _PT

        pltpu.sync_copy(idx_hbm.at[wid], idx_v)

        for u in range(_NBUF):
            pltpu.async_copy(
                table_hbm.at[idx_v.at[u]], rows_v.at[u], gsem.at[u]
            )

        n_groups = _PT // _NBUF

        def body(g, carry):
            for u in range(_NBUF):
                m = g * _NBUF + u
                pltpu.make_async_copy(
                    table_hbm.at[idx_v.at[m]], rows_v.at[u], gsem.at[u]
                ).wait()
                pltpu.async_copy(
                    rows_v.at[u], out_hbm.at[pl.ds((base + m) * _CH, _CH)], ssem.at[u]
                )

                @pl.when(g < n_groups - 1)
                def _():
                    pltpu.make_async_copy(
                        rows_v.at[u],
                        out_hbm.at[pl.ds((base + m) * _CH, _CH)],
                        ssem.at[u],
                    ).wait()
                    pltpu.async_copy(
                        table_hbm.at[idx_v.at[m + _NBUF]], rows_v.at[u], gsem.at[u]
                    )

            return carry

        lax.fori_loop(0, n_groups, body, 0)

        for u in range(_NBUF):
            m = _PT - _NBUF + u
            pltpu.make_async_copy(
                rows_v.at[u], out_hbm.at[pl.ds((base + m) * _CH, _CH)], ssem.at[u]
            ).wait()

    return gather_k(table, idx3)


def _edge_body(e_ref, vxg_ref, ux_ref, we_ref, be_ref, st_ref, out_ref):
    ve = (
        jnp.dot(e_ref[...], we_ref[...], preferred_element_type=jnp.float32)
        + be_ref[...]
    )
    ex = jnp.exp(ve)
    exv = (ex * vxg_ref[...].astype(jnp.float32)).astype(jnp.bfloat16)
    both = jnp.concatenate([ex.astype(jnp.bfloat16), exv], axis=1)
    sums = jnp.dot(st_ref[...], both, preferred_element_type=jnp.float32)
    h = out_ref.shape[1]
    out_ref[...] = ux_ref[...] + sums[:, h:] / sums[:, :h]


def _edge_pass(e2, vxg, ux, we_t, be, st, k):
    rows, h = e2.shape
    rb = _R * k
    grid = rows // rb
    return pl.pallas_call(
        _edge_body,
        grid=(grid,),
        in_specs=[
            pl.BlockSpec((rb, h), lambda i: (i, 0)),
            pl.BlockSpec((rb, h), lambda i: (i, 0)),
            pl.BlockSpec((_R, h), lambda i: (i, 0)),
            pl.BlockSpec((h, h), lambda i: (0, 0)),
            pl.BlockSpec((1, h), lambda i: (0, 0)),
            pl.BlockSpec((_R, rb), lambda i: (0, 0)),
        ],
        out_specs=pl.BlockSpec((_R, h), lambda i: (i, 0)),
        out_shape=jax.ShapeDtypeStruct((rows // k, h), jnp.float32),
    )(e2, vxg, ux, we_t, be, st)


def kernel(x, e, edge_index, W_node, b_node, W_to, b_to, W_edge, b_edge):
    b, n, h = x.shape
    nk = e.shape[1]
    k = nk // n

    x2 = x.reshape(b * n, h)
    e2 = e.reshape(b * nk, h)

    gidx = (
        edge_index.astype(jnp.int32) + (jnp.arange(b, dtype=jnp.int32) * n)[:, None]
    ).reshape(-1)
    idx3 = jnp.pad(gidx, (0, _TOT - b * nk)).reshape(_NW, _PT, _CH)

    ux, vx = _node_embeddings(x2, W_node.T, b_node[None], W_to.T, b_to[None])
    vxg = _sc_gather(vx, idx3)

    st = (jnp.arange(_R)[:, None] == (jnp.arange(_R * k) // k)[None, :]).astype(
        jnp.bfloat16
    )

    out2 = _edge_pass(e2, vxg, ux, W_edge.T, b_edge[None], st, k)
    return out2.reshape(b, n, h)


# speedup vs baseline: 1.3122x; 1.3122x over previous
import functools

import jax
import jax.numpy as jnp
from jax import lax
from jax.experimental import pallas as pl
from jax.experimental.pallas import tpu as pltpu
from jax.experimental.pallas import tpu_sc as plsc

_NC = 2
_NS = 16
_NW = _NC * _NS
_CH = 128
_PT = 50
_NBUF = 5
_TOTB = _NW * _PT * _CH

_R = 40


def _node_body(x_ref, wn_ref, bn_ref, wt_ref, bt_ref, ux_ref, vx_ref):
    xb = x_ref[...]
    ux_ref[...] = (
        jnp.dot(xb, wn_ref[...], preferred_element_type=jnp.float32) + bn_ref[...]
    )
    vx_ref[...] = (
        jnp.dot(xb, wt_ref[...], preferred_element_type=jnp.float32) + bt_ref[...]
    )


def _node_embeddings(x2, wn_t, bn, wt_t, bt):
    bn_rows, h = x2.shape
    blk = 2000 if bn_rows % 2000 == 0 else bn_rows
    grid = bn_rows // blk
    return pl.pallas_call(
        _node_body,
        grid=(grid,),
        in_specs=[
            pl.BlockSpec((blk, h), lambda i: (i, 0)),
            pl.BlockSpec((h, h), lambda i: (0, 0)),
            pl.BlockSpec((1, h), lambda i: (0, 0)),
            pl.BlockSpec((h, h), lambda i: (0, 0)),
            pl.BlockSpec((1, h), lambda i: (0, 0)),
        ],
        out_specs=[
            pl.BlockSpec((blk, h), lambda i: (i, 0)),
            pl.BlockSpec((blk, h), lambda i: (i, 0)),
        ],
        out_shape=[
            jax.ShapeDtypeStruct((bn_rows, h), jnp.float32),
            jax.ShapeDtypeStruct((bn_rows, h), jnp.float32),
        ],
    )(x2, wn_t, bn, wt_t, bt)


def _sc_gather(table, idx3):
    h = table.shape[1]
    mesh = plsc.VectorSubcoreMesh(
        core_axis_name="c", subcore_axis_name="s", num_cores=_NC, num_subcores=_NS
    )

    @functools.partial(
        pl.kernel,
        out_type=jax.ShapeDtypeStruct((_TOTB, h), jnp.float32),
        mesh=mesh,
        scratch_types=[
            pltpu.VMEM((_PT, _CH), jnp.int32),
            pltpu.VMEM((_NBUF, _CH, h), jnp.float32),
            pltpu.SemaphoreType.DMA((_NBUF,)),
            pltpu.SemaphoreType.DMA((_NBUF,)),
        ],
    )
    def gather_k(table_hbm, idx_hbm, out_hbm, idx_v, rows_v, gsem, ssem):
        wid = lax.axis_index("s") * _NC + lax.axis_index("c")
        base = wid * _PT

        pltpu.sync_copy(idx_hbm.at[wid], idx_v)

        n_groups = _PT // _NBUF

        def body(g, carry):
            m0 = g * _NBUF
            for u in range(_NBUF):
                pltpu.async_copy(
                    table_hbm.at[idx_v.at[m0 + u]], rows_v.at[u], gsem.at[u]
                )
            for u in range(_NBUF):
                pltpu.make_async_copy(
                    table_hbm.at[idx_v.at[m0 + u]], rows_v.at[u], gsem.at[u]
                ).wait()
                pltpu.async_copy(
                    rows_v.at[u],
                    out_hbm.at[pl.ds((base + m0 + u) * _CH, _CH)],
                    ssem.at[u],
                )
            for u in range(_NBUF):
                pltpu.make_async_copy(
                    rows_v.at[u],
                    out_hbm.at[pl.ds((base + m0 + u) * _CH, _CH)],
                    ssem.at[u],
                ).wait()
            return carry

        lax.fori_loop(0, n_groups, body, 0)

    return gather_k(table, idx3)


def _edge_body(e_ref, vxg_ref, ux_ref, we_ref, be_ref, st_ref, out_ref):
    ve = (
        jnp.dot(e_ref[...], we_ref[...], preferred_element_type=jnp.float32)
        + be_ref[...]
    )
    ex = jnp.exp(ve)
    exv = (ex * vxg_ref[...]).astype(jnp.bfloat16)
    both = jnp.concatenate([ex.astype(jnp.bfloat16), exv], axis=1)
    sums = jnp.dot(st_ref[...], both, preferred_element_type=jnp.float32)
    h = out_ref.shape[1]
    out_ref[...] = ux_ref[...] + sums[:, h:] / sums[:, :h]


def _edge_pass(e2, vxg, ux, we_t, be, st, k):
    rows, h = e2.shape
    rb = _R * k
    grid = rows // rb
    return pl.pallas_call(
        _edge_body,
        grid=(grid,),
        in_specs=[
            pl.BlockSpec((rb, h), lambda i: (i, 0)),
            pl.BlockSpec((rb, h), lambda i: (i, 0)),
            pl.BlockSpec((_R, h), lambda i: (i, 0)),
            pl.BlockSpec((h, h), lambda i: (0, 0)),
            pl.BlockSpec((1, h), lambda i: (0, 0)),
            pl.BlockSpec((_R, rb), lambda i: (0, 0)),
        ],
        out_specs=pl.BlockSpec((_R, h), lambda i: (i, 0)),
        out_shape=jax.ShapeDtypeStruct((rows // k, h), jnp.float32),
    )(e2, vxg, ux, we_t, be, st)


def kernel(x, e, edge_index, W_node, b_node, W_to, b_to, W_edge, b_edge):
    b, n, h = x.shape
    nk = e.shape[1]
    k = nk // n

    x2 = x.reshape(b * n, h)

    ux, vx = _node_embeddings(x2, W_node.T, b_node[None], W_to.T, b_to[None])

    st = (jnp.arange(_R)[:, None] == (jnp.arange(_R * k) // k)[None, :]).astype(
        jnp.bfloat16
    )

    outs = []
    for bi in range(b):
        idx3 = jnp.pad(edge_index[bi].astype(jnp.int32), (0, _TOTB - nk)).reshape(
            _NW, _PT, _CH
        )
        vxg = _sc_gather(lax.slice_in_dim(vx, bi * n, (bi + 1) * n), idx3)
        outs.append(
            _edge_pass(
                e[bi],
                vxg,
                lax.slice_in_dim(ux, bi * n, (bi + 1) * n),
                W_edge.T,
                b_edge[None],
                st,
                k,
            )
        )
    return jnp.stack(outs)


# speedup vs baseline: 2.1502x; 1.6386x over previous
import functools

import jax
import jax.numpy as jnp
from jax import lax
from jax.experimental import pallas as pl
from jax.experimental.pallas import tpu as pltpu
from jax.experimental.pallas import tpu_sc as plsc

_NC = 2
_NS = 16
_NW = _NC * _NS
_CH = 128
_PT = 50
_NBUF = 2
_TOTB = _NW * _PT * _CH

_R = 40


def _node_body(x_ref, wn_ref, bn_ref, wt_ref, bt_ref, ux_ref, vx_ref):
    xb = x_ref[...]
    ux_ref[...] = (
        jnp.dot(xb, wn_ref[...], preferred_element_type=jnp.float32) + bn_ref[...]
    )
    vx_ref[...] = (
        jnp.dot(xb, wt_ref[...], preferred_element_type=jnp.float32) + bt_ref[...]
    )


def _node_embeddings(x2, wn_t, bn, wt_t, bt):
    bn_rows, h = x2.shape
    blk = 2000 if bn_rows % 2000 == 0 else bn_rows
    grid = bn_rows // blk
    return pl.pallas_call(
        _node_body,
        grid=(grid,),
        in_specs=[
            pl.BlockSpec((blk, h), lambda i: (i, 0)),
            pl.BlockSpec((h, h), lambda i: (0, 0)),
            pl.BlockSpec((1, h), lambda i: (0, 0)),
            pl.BlockSpec((h, h), lambda i: (0, 0)),
            pl.BlockSpec((1, h), lambda i: (0, 0)),
        ],
        out_specs=[
            pl.BlockSpec((blk, h), lambda i: (i, 0)),
            pl.BlockSpec((blk, h), lambda i: (i, 0)),
        ],
        out_shape=[
            jax.ShapeDtypeStruct((bn_rows, h), jnp.float32),
            jax.ShapeDtypeStruct((bn_rows, h), jnp.float32),
        ],
    )(x2, wn_t, bn, wt_t, bt)


def _sc_gather(table, idx3):
    rows_t, h = table.shape
    stage = rows_t // _NS
    mesh = plsc.VectorSubcoreMesh(
        core_axis_name="c", subcore_axis_name="s", num_cores=_NC, num_subcores=_NS
    )

    @functools.partial(
        pl.kernel,
        out_type=jax.ShapeDtypeStruct((_TOTB, h), jnp.float32),
        mesh=mesh,
        scratch_types=[
            pltpu.VMEM((_PT, _CH), jnp.int32),
            pltpu.VMEM((_NBUF, _CH, h), jnp.float32),
            pltpu.VMEM_SHARED((rows_t, h), jnp.float32),
            pltpu.SemaphoreType.DMA((_NBUF,)),
            pltpu.SemaphoreType.DMA((_NBUF,)),
        ],
    )
    def gather_k(table_hbm, idx_hbm, out_hbm, idx_v, rows_v, tab_sh, gsem, ssem):
        cid = lax.axis_index("c")
        sid = lax.axis_index("s")
        wid = sid * _NC + cid
        base = wid * _PT

        pltpu.sync_copy(
            table_hbm.at[pl.ds(sid * stage, stage)],
            tab_sh.at[pl.ds(sid * stage, stage)],
        )
        pltpu.sync_copy(idx_hbm.at[wid], idx_v)
        plsc.subcore_barrier()

        n_groups = _PT // _NBUF

        def body(g, carry):
            m0 = g * _NBUF
            for u in range(_NBUF):
                pltpu.async_copy(
                    tab_sh.at[idx_v.at[m0 + u]], rows_v.at[u], gsem.at[u]
                )
            for u in range(_NBUF):
                pltpu.make_async_copy(
                    tab_sh.at[idx_v.at[m0 + u]], rows_v.at[u], gsem.at[u]
                ).wait()
                pltpu.async_copy(
                    rows_v.at[u],
                    out_hbm.at[pl.ds((base + m0 + u) * _CH, _CH)],
                    ssem.at[u],
                )
            for u in range(_NBUF):
                pltpu.make_async_copy(
                    rows_v.at[u],
                    out_hbm.at[pl.ds((base + m0 + u) * _CH, _CH)],
                    ssem.at[u],
                ).wait()
            return carry

        lax.fori_loop(0, n_groups, body, 0)

    return gather_k(table, idx3)


def _edge_body(e_ref, vxg_ref, ux_ref, we_ref, be_ref, st_ref, out_ref):
    ve = (
        jnp.dot(e_ref[...], we_ref[...], preferred_element_type=jnp.float32)
        + be_ref[...]
    )
    ex = jnp.exp(ve)
    exv = (ex * vxg_ref[...]).astype(jnp.bfloat16)
    both = jnp.concatenate([ex.astype(jnp.bfloat16), exv], axis=1)
    sums = jnp.dot(st_ref[...], both, preferred_element_type=jnp.float32)
    h = out_ref.shape[1]
    out_ref[...] = ux_ref[...] + sums[:, h:] / sums[:, :h]


def _edge_pass(e2, vxg, ux, we_t, be, st, k):
    rows, h = e2.shape
    rb = _R * k
    grid = rows // rb
    return pl.pallas_call(
        _edge_body,
        grid=(grid,),
        in_specs=[
            pl.BlockSpec((rb, h), lambda i: (i, 0)),
            pl.BlockSpec((rb, h), lambda i: (i, 0)),
            pl.BlockSpec((_R, h), lambda i: (i, 0)),
            pl.BlockSpec((h, h), lambda i: (0, 0)),
            pl.BlockSpec((1, h), lambda i: (0, 0)),
            pl.BlockSpec((_R, rb), lambda i: (0, 0)),
        ],
        out_specs=pl.BlockSpec((_R, h), lambda i: (i, 0)),
        out_shape=jax.ShapeDtypeStruct((rows // k, h), jnp.float32),
    )(e2, vxg, ux, we_t, be, st)


def kernel(x, e, edge_index, W_node, b_node, W_to, b_to, W_edge, b_edge):
    b, n, h = x.shape
    nk = e.shape[1]
    k = nk // n

    x2 = x.reshape(b * n, h)

    ux, vx = _node_embeddings(x2, W_node.T, b_node[None], W_to.T, b_to[None])

    st = (jnp.arange(_R)[:, None] == (jnp.arange(_R * k) // k)[None, :]).astype(
        jnp.bfloat16
    )

    outs = []
    for bi in range(b):
        idx3 = jnp.pad(edge_index[bi].astype(jnp.int32), (0, _TOTB - nk)).reshape(
            _NW, _PT, _CH
        )
        n_pad = -n % (8 * _NS)
        vxg = _sc_gather(
            jnp.pad(lax.slice_in_dim(vx, bi * n, (bi + 1) * n), ((0, n_pad), (0, 0))),
            idx3,
        )
        outs.append(
            _edge_pass(
                e[bi],
                vxg,
                lax.slice_in_dim(ux, bi * n, (bi + 1) * n),
                W_edge.T,
                b_edge[None],
                st,
                k,
            )
        )
    return jnp.stack(outs)


# speedup vs baseline: 2.4427x; 1.1360x over previous
import functools

import jax
import jax.numpy as jnp
from jax import lax
from jax.experimental import pallas as pl
from jax.experimental.pallas import tpu as pltpu
from jax.experimental.pallas import tpu_sc as plsc

_NC = 2
_NS = 16
_NW = _NC * _NS
_CH = 128
_PTB = 50
_NBUF = 2
_TOTB = _NW * _PTB * _CH

_R = 40


def _node_body(x_ref, wn_ref, bn_ref, wt_ref, bt_ref, ux_ref, vx_ref):
    xb = x_ref[...]
    ux_ref[...] = (
        jnp.dot(xb, wn_ref[...], preferred_element_type=jnp.float32) + bn_ref[...]
    )
    vx_ref[...] = (
        jnp.dot(xb, wt_ref[...], preferred_element_type=jnp.float32) + bt_ref[...]
    )


def _node_embeddings(x2, wn_t, bn, wt_t, bt):
    bn_rows, h = x2.shape
    blk = 2000 if bn_rows % 2000 == 0 else bn_rows
    grid = bn_rows // blk
    return pl.pallas_call(
        _node_body,
        grid=(grid,),
        in_specs=[
            pl.BlockSpec((blk, h), lambda i: (i, 0)),
            pl.BlockSpec((h, h), lambda i: (0, 0)),
            pl.BlockSpec((1, h), lambda i: (0, 0)),
            pl.BlockSpec((h, h), lambda i: (0, 0)),
            pl.BlockSpec((1, h), lambda i: (0, 0)),
        ],
        out_specs=[
            pl.BlockSpec((blk, h), lambda i: (i, 0)),
            pl.BlockSpec((blk, h), lambda i: (i, 0)),
        ],
        out_shape=[
            jax.ShapeDtypeStruct((bn_rows, h), jnp.float32),
            jax.ShapeDtypeStruct((bn_rows, h), jnp.float32),
        ],
    )(x2, wn_t, bn, wt_t, bt)


def _sc_gather(tables, idx3):
    nb, rows_t, h = tables.shape
    stage = rows_t // _NS
    mesh = plsc.VectorSubcoreMesh(
        core_axis_name="c", subcore_axis_name="s", num_cores=_NC, num_subcores=_NS
    )

    @functools.partial(
        pl.kernel,
        out_type=jax.ShapeDtypeStruct((nb * _TOTB, h), jnp.float32),
        mesh=mesh,
        scratch_types=[
            pltpu.VMEM((nb * _PTB, _CH), jnp.int32),
            pltpu.VMEM((_NBUF, _CH, h), jnp.float32),
            pltpu.VMEM_SHARED((rows_t, h), jnp.float32),
            pltpu.SemaphoreType.DMA((_NBUF,)),
            pltpu.SemaphoreType.DMA((_NBUF,)),
        ],
    )
    def gather_k(tab_hbm, idx_hbm, out_hbm, idx_v, rows_v, tab_sh, gsem, ssem):
        cid = lax.axis_index("c")
        sid = lax.axis_index("s")
        wid = sid * _NC + cid

        pltpu.sync_copy(idx_hbm.at[wid], idx_v)

        n_groups = _PTB // _NBUF

        for phase in range(nb):
            pltpu.sync_copy(
                tab_hbm.at[phase, pl.ds(sid * stage, stage)],
                tab_sh.at[pl.ds(sid * stage, stage)],
            )
            plsc.subcore_barrier()

            chunk0 = phase * _NW * _PTB + wid * _PTB

            def body(g, carry):
                m0 = g * _NBUF
                for u in range(_NBUF):
                    pltpu.async_copy(
                        tab_sh.at[idx_v.at[phase * _PTB + m0 + u]],
                        rows_v.at[u],
                        gsem.at[u],
                    )
                for u in range(_NBUF):
                    pltpu.make_async_copy(
                        tab_sh.at[idx_v.at[phase * _PTB + m0 + u]],
                        rows_v.at[u],
                        gsem.at[u],
                    ).wait()
                    pltpu.async_copy(
                        rows_v.at[u],
                        out_hbm.at[pl.ds((chunk0 + m0 + u) * _CH, _CH)],
                        ssem.at[u],
                    )
                for u in range(_NBUF):
                    pltpu.make_async_copy(
                        rows_v.at[u],
                        out_hbm.at[pl.ds((chunk0 + m0 + u) * _CH, _CH)],
                        ssem.at[u],
                    ).wait()
                return carry

            lax.fori_loop(0, n_groups, body, 0)
            plsc.subcore_barrier()

    return gather_k(tables, idx3)


def _edge_body(e_ref, vxg_ref, ux_ref, we_ref, be_ref, st_ref, out_ref):
    ve = (
        jnp.dot(e_ref[...], we_ref[...], preferred_element_type=jnp.float32)
        + be_ref[...]
    )
    ex = jnp.exp(ve)
    exv = (ex * vxg_ref[...]).astype(jnp.bfloat16)
    both = jnp.concatenate([ex.astype(jnp.bfloat16), exv], axis=1)
    sums = jnp.dot(st_ref[...], both, preferred_element_type=jnp.float32)
    h = out_ref.shape[1]
    out_ref[...] = ux_ref[...] + sums[:, h:] / sums[:, :h]


def _edge_pass(e2, vxg, ux, we_t, be, st, n, k):
    rows, h = e2.shape
    rb = _R * k
    grid = rows // rb
    bpb = n // _R
    pad_blk = (_TOTB - n * k) // rb

    return pl.pallas_call(
        _edge_body,
        grid=(grid,),
        in_specs=[
            pl.BlockSpec((rb, h), lambda i: (i, 0)),
            pl.BlockSpec((rb, h), lambda i: (i + (i // bpb) * pad_blk, 0)),
            pl.BlockSpec((_R, h), lambda i: (i, 0)),
            pl.BlockSpec((h, h), lambda i: (0, 0)),
            pl.BlockSpec((1, h), lambda i: (0, 0)),
            pl.BlockSpec((_R, rb), lambda i: (0, 0)),
        ],
        out_specs=pl.BlockSpec((_R, h), lambda i: (i, 0)),
        out_shape=jax.ShapeDtypeStruct((rows // k, h), jnp.float32),
    )(e2, vxg, ux, we_t, be, st)


def kernel(x, e, edge_index, W_node, b_node, W_to, b_to, W_edge, b_edge):
    b, n, h = x.shape
    nk = e.shape[1]
    k = nk // n

    x2 = x.reshape(b * n, h)
    e2 = e.reshape(b * nk, h)

    ux, vx = _node_embeddings(x2, W_node.T, b_node[None], W_to.T, b_to[None])

    n_pad = -n % (8 * _NS)
    tables = jnp.pad(vx.reshape(b, n, h), ((0, 0), (0, n_pad), (0, 0)))

    idx3 = jnp.concatenate(
        [
            jnp.pad(edge_index[bi].astype(jnp.int32), (0, _TOTB - nk)).reshape(
                _NW, _PTB, _CH
            )
            for bi in range(b)
        ],
        axis=1,
    )

    vxg = _sc_gather(tables, idx3)

    st = (jnp.arange(_R)[:, None] == (jnp.arange(_R * k) // k)[None, :]).astype(
        jnp.bfloat16
    )

    out2 = _edge_pass(e2, vxg, ux, W_edge.T, b_edge[None], st, n, k)
    return out2.reshape(b, n, h)


# speedup vs baseline: 2.4660x; 1.0096x over previous
import functools

import jax
import jax.numpy as jnp
from jax import lax
from jax.experimental import pallas as pl
from jax.experimental.pallas import tpu as pltpu
from jax.experimental.pallas import tpu_sc as plsc

_NC = 2
_NS = 16
_NW = _NC * _NS
_CH = 128
_PTB = 50
_NBUF = 2
_TOTB = _NW * _PTB * _CH

_R = 40


def _dot_t(a, w):
    return lax.dot_general(
        a, w, (((1,), (1,)), ((), ())), preferred_element_type=jnp.float32
    )


def _node_body(x_ref, wn_ref, bn_ref, wt_ref, bt_ref, ux_ref, vx_ref):
    xb = x_ref[...]
    ux_ref[...] = _dot_t(xb, wn_ref[...]) + bn_ref[...]
    vx_ref[...] = _dot_t(xb, wt_ref[...]) + bt_ref[...]


def _node_embeddings(x2, wn_t, bn, wt_t, bt):
    bn_rows, h = x2.shape
    blk = 2000 if bn_rows % 2000 == 0 else bn_rows
    grid = bn_rows // blk
    return pl.pallas_call(
        _node_body,
        grid=(grid,),
        in_specs=[
            pl.BlockSpec((blk, h), lambda i: (i, 0)),
            pl.BlockSpec((h, h), lambda i: (0, 0)),
            pl.BlockSpec((1, h), lambda i: (0, 0)),
            pl.BlockSpec((h, h), lambda i: (0, 0)),
            pl.BlockSpec((1, h), lambda i: (0, 0)),
        ],
        out_specs=[
            pl.BlockSpec((blk, h), lambda i: (i, 0)),
            pl.BlockSpec((blk, h), lambda i: (i, 0)),
        ],
        out_shape=[
            jax.ShapeDtypeStruct((bn_rows, h), jnp.float32),
            jax.ShapeDtypeStruct((bn_rows, h), jnp.float32),
        ],
    )(x2, wn_t, bn, wt_t, bt)


def _sc_gather(tables, idx3):
    nb, rows_t, h = tables.shape
    stage = (rows_t // _NS) // 8 * 8
    rem = rows_t - stage * _NS
    mesh = plsc.VectorSubcoreMesh(
        core_axis_name="c", subcore_axis_name="s", num_cores=_NC, num_subcores=_NS
    )

    @functools.partial(
        pl.kernel,
        out_type=jax.ShapeDtypeStruct((nb * _TOTB, h), jnp.float32),
        mesh=mesh,
        scratch_types=[
            pltpu.VMEM((nb * _PTB, _CH), jnp.int32),
            pltpu.VMEM((_NBUF, _CH, h), jnp.float32),
            pltpu.VMEM_SHARED((rows_t, h), jnp.float32),
            pltpu.SemaphoreType.DMA((_NBUF,)),
            pltpu.SemaphoreType.DMA((_NBUF,)),
        ],
    )
    def gather_k(tab_hbm, idx_hbm, out_hbm, idx_v, rows_v, tab_sh, gsem, ssem):
        cid = lax.axis_index("c")
        sid = lax.axis_index("s")
        wid = sid * _NC + cid

        pltpu.sync_copy(idx_hbm.at[wid], idx_v)

        n_groups = _PTB // _NBUF

        for phase in range(nb):
            pltpu.sync_copy(
                tab_hbm.at[phase, pl.ds(sid * stage, stage)],
                tab_sh.at[pl.ds(sid * stage, stage)],
            )
            if rem:

                @pl.when(sid == _NS - 1)
                def _():
                    pltpu.sync_copy(
                        tab_hbm.at[phase, pl.ds(stage * _NS, rem)],
                        tab_sh.at[pl.ds(stage * _NS, rem)],
                    )

            plsc.subcore_barrier()

            chunk0 = phase * _NW * _PTB + wid * _PTB

            def body(g, carry):
                m0 = g * _NBUF
                for u in range(_NBUF):
                    pltpu.async_copy(
                        tab_sh.at[idx_v.at[phase * _PTB + m0 + u]],
                        rows_v.at[u],
                        gsem.at[u],
                    )
                for u in range(_NBUF):
                    pltpu.make_async_copy(
                        tab_sh.at[idx_v.at[phase * _PTB + m0 + u]],
                        rows_v.at[u],
                        gsem.at[u],
                    ).wait()
                    pltpu.async_copy(
                        rows_v.at[u],
                        out_hbm.at[pl.ds((chunk0 + m0 + u) * _CH, _CH)],
                        ssem.at[u],
                    )
                for u in range(_NBUF):
                    pltpu.make_async_copy(
                        rows_v.at[u],
                        out_hbm.at[pl.ds((chunk0 + m0 + u) * _CH, _CH)],
                        ssem.at[u],
                    ).wait()
                return carry

            lax.fori_loop(0, n_groups, body, 0)
            plsc.subcore_barrier()

    return gather_k(tables, idx3)


def _make_edge_body(k):
    rb = _R * k

    def _edge_body(e_ref, vxg_ref, ux_ref, we_ref, be_ref, out_ref):
        ve = _dot_t(e_ref[...], we_ref[...]) + be_ref[...]
        ex = jnp.exp(ve)
        exv = (ex * vxg_ref[...]).astype(jnp.bfloat16)
        both = jnp.concatenate([ex.astype(jnp.bfloat16), exv], axis=1)
        row = lax.broadcasted_iota(jnp.int32, (_R, rb), 0)
        col = lax.broadcasted_iota(jnp.int32, (_R, rb), 1)
        st = ((col >= row * k) & (col < (row + 1) * k)).astype(jnp.bfloat16)
        sums = jnp.dot(st, both, preferred_element_type=jnp.float32)
        h = out_ref.shape[1]
        out_ref[...] = ux_ref[...] + sums[:, h:] / sums[:, :h]

    return _edge_body


def _edge_pass(e2, vxg, ux, we, be, n, k):
    rows, h = e2.shape
    rb = _R * k
    grid = rows // rb
    bpb = n // _R
    pad_blk = (_TOTB - n * k) // rb

    return pl.pallas_call(
        _make_edge_body(k),
        grid=(grid,),
        in_specs=[
            pl.BlockSpec((rb, h), lambda i: (i, 0)),
            pl.BlockSpec((rb, h), lambda i: (i + (i // bpb) * pad_blk, 0)),
            pl.BlockSpec((_R, h), lambda i: (i, 0)),
            pl.BlockSpec((h, h), lambda i: (0, 0)),
            pl.BlockSpec((1, h), lambda i: (0, 0)),
        ],
        out_specs=pl.BlockSpec((_R, h), lambda i: (i, 0)),
        out_shape=jax.ShapeDtypeStruct((rows // k, h), jnp.float32),
    )(e2, vxg, ux, we, be)


def kernel(x, e, edge_index, W_node, b_node, W_to, b_to, W_edge, b_edge):
    b, n, h = x.shape
    nk = e.shape[1]
    k = nk // n

    x2 = x.reshape(b * n, h)
    e2 = e.reshape(b * nk, h)

    ux, vx = _node_embeddings(x2, W_node, b_node[None], W_to, b_to[None])
    tables = vx.reshape(b, n, h)

    idx3 = jnp.concatenate(
        [
            jnp.pad(edge_index[bi].astype(jnp.int32), (0, _TOTB - nk)).reshape(
                _NW, _PTB, _CH
            )
            for bi in range(b)
        ],
        axis=1,
    )

    vxg = _sc_gather(tables, idx3)
    out2 = _edge_pass(e2, vxg, ux, W_edge, b_edge[None], n, k)
    return out2.reshape(b, n, h)


# speedup vs baseline: 3.2518x; 1.3187x over previous
import functools

import jax
import jax.numpy as jnp
from jax import lax
from jax.experimental import pallas as pl
from jax.experimental.pallas import tpu as pltpu
from jax.experimental.pallas import tpu_sc as plsc

_NC = 2
_NS = 16
_NW = _NC * _NS
_CH = 128
_PTB = 50
_NBUF = 2
_TOTB = _NW * _PTB * _CH

_R = 80


def _dot_t(a, w):
    return lax.dot_general(
        a, w, (((1,), (1,)), ((), ())), preferred_element_type=jnp.float32
    )


def _node_body(x_ref, wn_ref, bn_ref, wt_ref, bt_ref, ux_ref, vx_ref):
    xb = x_ref[...]
    ux_ref[...] = _dot_t(xb, wn_ref[...]) + bn_ref[...]
    vx_ref[...] = _dot_t(xb, wt_ref[...]) + bt_ref[...]


def _node_embeddings(x2, wn_t, bn, wt_t, bt):
    bn_rows, h = x2.shape
    blk = 2000 if bn_rows % 2000 == 0 else bn_rows
    grid = bn_rows // blk
    return pl.pallas_call(
        _node_body,
        grid=(grid,),
        in_specs=[
            pl.BlockSpec((blk, h), lambda i: (i, 0)),
            pl.BlockSpec((h, h), lambda i: (0, 0)),
            pl.BlockSpec((1, h), lambda i: (0, 0)),
            pl.BlockSpec((h, h), lambda i: (0, 0)),
            pl.BlockSpec((1, h), lambda i: (0, 0)),
        ],
        out_specs=[
            pl.BlockSpec((blk, h), lambda i: (i, 0)),
            pl.BlockSpec((blk, h), lambda i: (i, 0)),
        ],
        out_shape=[
            jax.ShapeDtypeStruct((bn_rows, h), jnp.float32),
            jax.ShapeDtypeStruct((bn_rows, h), jnp.float32),
        ],
    )(x2, wn_t, bn, wt_t, bt)


def _sc_gather(tables, idx3):
    nb, rows_t, h = tables.shape
    stage = (rows_t // _NS) // 8 * 8
    rem = rows_t - stage * _NS
    mesh = plsc.VectorSubcoreMesh(
        core_axis_name="c", subcore_axis_name="s", num_cores=_NC, num_subcores=_NS
    )

    @functools.partial(
        pl.kernel,
        out_type=jax.ShapeDtypeStruct((nb * _TOTB, h), jnp.float32),
        mesh=mesh,
        scratch_types=[
            pltpu.VMEM((nb * _PTB, _CH), jnp.int32),
            pltpu.VMEM((_NBUF, _CH, h), jnp.float32),
            pltpu.VMEM_SHARED((rows_t, h), jnp.float32),
            pltpu.SemaphoreType.DMA((_NBUF,)),
            pltpu.SemaphoreType.DMA((_NBUF,)),
        ],
    )
    def gather_k(tab_hbm, idx_hbm, out_hbm, idx_v, rows_v, tab_sh, gsem, ssem):
        cid = lax.axis_index("c")
        sid = lax.axis_index("s")
        wid = sid * _NC + cid

        pltpu.sync_copy(idx_hbm.at[wid], idx_v)

        n_groups = _PTB // _NBUF

        for phase in range(nb):
            pltpu.sync_copy(
                tab_hbm.at[phase, pl.ds(sid * stage, stage)],
                tab_sh.at[pl.ds(sid * stage, stage)],
            )
            if rem:

                @pl.when(sid == _NS - 1)
                def _():
                    pltpu.sync_copy(
                        tab_hbm.at[phase, pl.ds(stage * _NS, rem)],
                        tab_sh.at[pl.ds(stage * _NS, rem)],
                    )

            plsc.subcore_barrier()

            chunk0 = phase * _NW * _PTB + wid * _PTB

            def body(g, carry):
                m0 = g * _NBUF
                for u in range(_NBUF):
                    pltpu.async_copy(
                        tab_sh.at[idx_v.at[phase * _PTB + m0 + u]],
                        rows_v.at[u],
                        gsem.at[u],
                    )
                for u in range(_NBUF):
                    pltpu.make_async_copy(
                        tab_sh.at[idx_v.at[phase * _PTB + m0 + u]],
                        rows_v.at[u],
                        gsem.at[u],
                    ).wait()
                    pltpu.async_copy(
                        rows_v.at[u],
                        out_hbm.at[pl.ds((chunk0 + m0 + u) * _CH, _CH)],
                        ssem.at[u],
                    )
                for u in range(_NBUF):
                    pltpu.make_async_copy(
                        rows_v.at[u],
                        out_hbm.at[pl.ds((chunk0 + m0 + u) * _CH, _CH)],
                        ssem.at[u],
                    ).wait()
                return carry

            lax.fori_loop(0, n_groups, body, 0)
            plsc.subcore_barrier()

    return gather_k(tables, idx3)


def _make_edge_body(k):
    rb = _R * k

    def _edge_body(e_ref, vxg_ref, ux_ref, we_ref, be_ref, out_ref):
        ve = _dot_t(e_ref[...], we_ref[...]) + be_ref[...]
        ex = jnp.exp(ve)
        exv = (ex * vxg_ref[...]).astype(jnp.bfloat16)
        both = jnp.concatenate([ex.astype(jnp.bfloat16), exv], axis=1)
        row = lax.broadcasted_iota(jnp.int32, (_R, rb), 0)
        col = lax.broadcasted_iota(jnp.int32, (_R, rb), 1)
        st = ((col >= row * k) & (col < (row + 1) * k)).astype(jnp.bfloat16)
        sums = jnp.dot(st, both, preferred_element_type=jnp.float32)
        h = out_ref.shape[1]
        out_ref[...] = ux_ref[...] + sums[:, h:] / sums[:, :h]

    return _edge_body


def _edge_pass(e2, vxg, ux, we, be, n, k):
    rows, h = e2.shape
    rb = _R * k
    grid = rows // rb
    bpb = n // _R
    pad_blk = (_TOTB - n * k) // rb

    return pl.pallas_call(
        _make_edge_body(k),
        grid=(grid,),
        in_specs=[
            pl.BlockSpec((rb, h), lambda i: (i, 0)),
            pl.BlockSpec((rb, h), lambda i: (i + (i // bpb) * pad_blk, 0)),
            pl.BlockSpec((_R, h), lambda i: (i, 0)),
            pl.BlockSpec((h, h), lambda i: (0, 0)),
            pl.BlockSpec((1, h), lambda i: (0, 0)),
        ],
        out_specs=pl.BlockSpec((_R, h), lambda i: (i, 0)),
        out_shape=jax.ShapeDtypeStruct((rows // k, h), jnp.float32),
    )(e2, vxg, ux, we, be)


def kernel(x, e, edge_index, W_node, b_node, W_to, b_to, W_edge, b_edge):
    b, n, h = x.shape
    nk = e.shape[1]
    k = nk // n

    x2 = x.reshape(b * n, h)
    e2 = e.reshape(b * nk, h)

    ux, vx = _node_embeddings(x2, W_node, b_node[None], W_to, b_to[None])
    tables = vx.reshape(b, n, h)

    idx3 = jnp.concatenate(
        [
            jnp.pad(edge_index[bi].astype(jnp.int32), (0, _TOTB - nk)).reshape(
                _NW, _PTB, _CH
            )
            for bi in range(b)
        ],
        axis=1,
    )

    vxg = _sc_gather(tables, idx3)
    out2 = _edge_pass(e2, vxg, ux, W_edge, b_edge[None], n, k)
    return out2.reshape(b, n, h)
